# baseline (speedup 1.0000x reference)
import functools

import jax
import jax.numpy as jnp
from jax import lax
from jax.experimental import pallas as pl
from jax.experimental.pallas import tpu as pltpu
from jax.experimental.pallas import tpu_sc as plsc

D = 128
B = 4096 * 200
CHUNK = 128
SLOTS = 4

_info = plsc.get_sparse_core_info()
NC, NS = _info.num_cores, _info.num_subcores
NW = NC * NS

SC_ROWS = 393216
TC_ROWS = B - SC_ROWS
B_PER_W = SC_ROWS // NW
N_CHUNKS = B_PER_W // CHUNK
N_GROUPS = N_CHUNKS // SLOTS

BR = 2048

_mesh = plsc.VectorSubcoreMesh(core_axis_name="c", subcore_axis_name="s")


@functools.partial(
    pl.kernel,
    mesh=_mesh,
    out_type=jax.ShapeDtypeStruct((SC_ROWS, D), jnp.float32),
    scratch_types=[
        pltpu.VMEM((N_CHUNKS, CHUNK), jnp.int32),
        pltpu.VMEM_SHARED((3, D), jnp.float32),
        pltpu.VMEM((SLOTS, CHUNK, D), jnp.float32),
        pltpu.SemaphoreType.DMA,
        pltpu.SemaphoreType.DMA,
    ],
)
def _emb_sc(x_hbm, table_hbm, out_hbm, idx_all, table_v, rows, gsem, osem):
    wid = lax.axis_index("s") * NC + lax.axis_index("c")
    row_base = wid * B_PER_W
    chunk_base = wid * N_CHUNKS

    pltpu.sync_copy(x_hbm.at[pl.ds(chunk_base, N_CHUNKS)], idx_all)

    @pl.when(lax.axis_index("s") == 0)
    def _():
        pltpu.sync_copy(table_hbm, table_v)

    plsc.subcore_barrier()

    def drain_out():
        pltpu.make_async_copy(
            rows.at[0], out_hbm.at[pl.ds(row_base, CHUNK)], osem).wait()

    def body(g, _):
        copies = []
        for b in range(SLOTS):
            j = g * SLOTS + b

            @pl.when(g >= 1)
            def _():
                drain_out()

            copies.append(pltpu.async_copy(
                table_v.at[idx_all.at[j]], rows.at[b], gsem))
        for b in range(SLOTS):
            j = g * SLOTS + b
            copies[b].wait()
            pltpu.make_async_copy(
                rows.at[b],
                out_hbm.at[pl.ds(row_base + j * CHUNK, CHUNK)],
                osem).start()
        return 0

    lax.fori_loop(0, N_GROUPS, body, 0)

    for _ in range(SLOTS):
        drain_out()


def _emb_tc_body(x_ref, t_ref, o_ref):
    xb = jnp.broadcast_to(x_ref[...], (BR, D))
    t = t_ref[...]
    t0 = jnp.broadcast_to(t[0:1, :], (BR, D))
    t1 = jnp.broadcast_to(t[1:2, :], (BR, D))
    t2 = jnp.broadcast_to(t[2:3, :], (BR, D))
    o_ref[...] = jnp.where(xb == 0, t0, jnp.where(xb == 1, t1, t2))


_emb_tc = pl.pallas_call(
    _emb_tc_body,
    grid=(TC_ROWS // BR,),
    in_specs=[
        pl.BlockSpec((BR, 1), lambda i: (i, 0)),
        pl.BlockSpec((3, D), lambda i: (0, 0)),
    ],
    out_specs=pl.BlockSpec((BR, D), lambda i: (i, 0)),
    out_shape=jax.ShapeDtypeStruct((TC_ROWS, D), jnp.float32),
)


def kernel(x, table):
    x1d = x.reshape(-1).astype(jnp.int32)
    x_sc = x1d[:SC_ROWS].reshape(-1, CHUNK)
    x_tc = x1d[SC_ROWS:].reshape(-1, 1)
    out_sc = _emb_sc(x_sc, table)
    out_tc = _emb_tc(x_tc, table)
    out = jnp.concatenate([out_sc, out_tc], axis=0)
    return out.reshape(x.shape + (D,))


# speedup vs baseline: 1.6079x; 1.6079x over previous
import functools

import jax
import jax.numpy as jnp
from jax import lax
from jax.experimental import pallas as pl
from jax.experimental.pallas import tpu as pltpu
from jax.experimental.pallas import tpu_sc as plsc

D = 128
B = 4096 * 200
CHUNK = 128
SLOTS = 4

_info = plsc.get_sparse_core_info()
NC, NS = _info.num_cores, _info.num_subcores
NW = NC * NS

SC_ROWS = 393216
TC_ROWS = B - SC_ROWS
B_PER_W = SC_ROWS // NW
N_CHUNKS = B_PER_W // CHUNK
N_GROUPS = N_CHUNKS // SLOTS

BR = 2048

_mesh = plsc.VectorSubcoreMesh(core_axis_name="c", subcore_axis_name="s")


@functools.partial(
    pl.kernel,
    mesh=_mesh,
    out_type=jax.ShapeDtypeStruct((B, D), jnp.float32),
    scratch_types=[
        pltpu.VMEM((N_CHUNKS, CHUNK), jnp.int32),
        pltpu.VMEM_SHARED((3, D), jnp.float32),
        pltpu.VMEM((SLOTS, CHUNK, D), jnp.float32),
        pltpu.SemaphoreType.DMA,
        pltpu.SemaphoreType.DMA,
    ],
)
def _emb_sc(x_hbm, table_hbm, out_hbm, idx_all, table_v, rows, gsem, osem):
    wid = lax.axis_index("s") * NC + lax.axis_index("c")
    row_base = wid * B_PER_W
    chunk_base = wid * N_CHUNKS

    pltpu.sync_copy(x_hbm.at[pl.ds(chunk_base, N_CHUNKS)], idx_all)

    @pl.when(lax.axis_index("s") == 0)
    def _():
        pltpu.sync_copy(table_hbm, table_v)

    plsc.subcore_barrier()

    def drain_out():
        pltpu.make_async_copy(
            rows.at[0], out_hbm.at[pl.ds(row_base, CHUNK)], osem).wait()

    def body(g, _):
        copies = []
        for b in range(SLOTS):
            j = g * SLOTS + b

            @pl.when(g >= 1)
            def _():
                drain_out()

            copies.append(pltpu.async_copy(
                table_v.at[idx_all.at[j]], rows.at[b], gsem))
        for b in range(SLOTS):
            j = g * SLOTS + b
            copies[b].wait()
            pltpu.make_async_copy(
                rows.at[b],
                out_hbm.at[pl.ds(row_base + j * CHUNK, CHUNK)],
                osem).start()
        return 0

    lax.fori_loop(0, N_GROUPS, body, 0)

    for _ in range(SLOTS):
        drain_out()


def _emb_tc_body(x_ref, t_ref, prev_ref, o_ref):
    del prev_ref
    xb = jnp.broadcast_to(x_ref[...], (BR, D))
    t = t_ref[...]
    t0 = jnp.broadcast_to(t[0:1, :], (BR, D))
    t1 = jnp.broadcast_to(t[1:2, :], (BR, D))
    t2 = jnp.broadcast_to(t[2:3, :], (BR, D))
    o_ref[...] = jnp.where(xb == 0, t0, jnp.where(xb == 1, t1, t2))


_TC_OFF = SC_ROWS // BR

_emb_tc = pl.pallas_call(
    _emb_tc_body,
    grid=(TC_ROWS // BR,),
    in_specs=[
        pl.BlockSpec((BR, 1), lambda i: (i, 0)),
        pl.BlockSpec((3, D), lambda i: (0, 0)),
        pl.BlockSpec(memory_space=pltpu.MemorySpace.HBM),
    ],
    out_specs=pl.BlockSpec((BR, D), lambda i: (i + _TC_OFF, 0)),
    out_shape=jax.ShapeDtypeStruct((B, D), jnp.float32),
    input_output_aliases={2: 0},
)


def kernel(x, table):
    x1d = x.reshape(-1).astype(jnp.int32)
    x_sc = x1d[:SC_ROWS].reshape(-1, CHUNK)
    x_tc = x1d[SC_ROWS:].reshape(-1, 1)
    out_sc = _emb_sc(x_sc, table)
    out = _emb_tc(x_tc, table, out_sc)
    return out.reshape(x.shape + (D,))


# speedup vs baseline: 3.2239x; 2.0050x over previous
import functools

import jax
import jax.numpy as jnp
from jax import lax
from jax.experimental import pallas as pl
from jax.experimental.pallas import tpu as pltpu
from jax.experimental.pallas import tpu_sc as plsc

D = 128
B = 4096 * 200
CHUNK = 128
SLOTS = 4

_info = plsc.get_sparse_core_info()
NC, NS = _info.num_cores, _info.num_subcores
NW = NC * NS
B_PER_W = B // NW
N_CHUNKS = B_PER_W // CHUNK
N_GROUPS = N_CHUNKS // SLOTS

_mesh = plsc.VectorSubcoreMesh(core_axis_name="c", subcore_axis_name="s")


@functools.partial(
    pl.kernel,
    mesh=_mesh,
    out_type=jax.ShapeDtypeStruct((B, D), jnp.float32),
    scratch_types=[
        pltpu.VMEM((N_CHUNKS, CHUNK), jnp.int32),
        pltpu.VMEM_SHARED((3, D), jnp.float32),
        pltpu.VMEM((SLOTS, CHUNK, D), jnp.float32),
        pltpu.SemaphoreType.DMA,
        pltpu.SemaphoreType.DMA,
    ],
)
def _emb_lookup(x_hbm, table_hbm, out_hbm, idx_all, table_v, rows, gsem,
                osem):
    wid = lax.axis_index("s") * NC + lax.axis_index("c")
    row_base = wid * B_PER_W
    chunk_base = wid * N_CHUNKS

    pltpu.sync_copy(x_hbm.at[pl.ds(chunk_base, N_CHUNKS)], idx_all)

    @pl.when(lax.axis_index("s") == 0)
    def _():
        pltpu.sync_copy(table_hbm, table_v)

    plsc.subcore_barrier()

    def drain_out():
        pltpu.make_async_copy(
            rows.at[0], out_hbm.at[pl.ds(row_base, CHUNK)], osem).wait()

    def body(g, _):
        copies = []
        for b in range(SLOTS):
            j = g * SLOTS + b

            @pl.when(g >= 1)
            def _():
                drain_out()

            copies.append(pltpu.async_copy(
                table_v.at[idx_all.at[j]], rows.at[b], gsem))
        for b in range(SLOTS):
            j = g * SLOTS + b
            copies[b].wait()
            pltpu.make_async_copy(
                rows.at[b],
                out_hbm.at[pl.ds(row_base + j * CHUNK, CHUNK)],
                osem).start()
        return 0

    lax.fori_loop(0, N_GROUPS, body, 0)

    for _ in range(SLOTS):
        drain_out()


def kernel(x, table):
    x2d = x.reshape(-1, CHUNK).astype(jnp.int32)
    out = _emb_lookup(x2d, table)
    return out.reshape(x.shape + (D,))
